# Initial kernel scaffold; baseline (speedup 1.0000x reference)
import jax
import jax.numpy as jnp
from jax.experimental import pallas as pl


def kernel(x, table):
    raise NotImplementedError("write your pallas kernel here")


# speedup vs baseline: 2.2074x; 2.2074x over previous
import functools

import jax
import jax.numpy as jnp
from jax import lax
from jax.experimental import pallas as pl
from jax.experimental.pallas import tpu as pltpu
from jax.experimental.pallas import tpu_sc as plsc

L = 16
IDX_MINOR = 128
CHUNK = 1024
D = 64


@functools.cache
def _build(B):
    info = plsc.get_sparse_core_info()
    nw = info.num_cores * info.num_subcores
    b_per_w = B // nw
    n_chunks = b_per_w // CHUNK
    rows2 = CHUNK // IDX_MINOR
    mesh = plsc.VectorSubcoreMesh(core_axis_name="c", subcore_axis_name="s")

    @functools.partial(
        pl.kernel,
        mesh=mesh,
        out_type=[
            jax.ShapeDtypeStruct((B, D), jnp.float32),
            jax.ShapeDtypeStruct((B // IDX_MINOR, IDX_MINOR), jnp.float32),
        ],
        scratch_types=[
            pltpu.VMEM((rows2, IDX_MINOR), jnp.int32),
            pltpu.VMEM((CHUNK, D), jnp.float32),
            pltpu.VMEM((rows2, IDX_MINOR), jnp.float32),
            pltpu.SemaphoreType.DMA,
        ],
        compiler_params=pltpu.CompilerParams(use_tc_tiling_on_sc=False),
    )
    def emb_kernel(x_hbm, table_hbm, emb_hbm, mask_hbm, idx_v, rows_v, mask_v, sem):
        wid = lax.axis_index("s") * info.num_cores + lax.axis_index("c")
        w_base = wid * b_per_w

        for t in range(n_chunks):
            base = pl.multiple_of(w_base + t * CHUNK, CHUNK)
            rb = pl.multiple_of(base // IDX_MINOR, CHUNK // IDX_MINOR)
            pltpu.sync_copy(x_hbm.at[pl.ds(rb, rows2)], idx_v)
            copies = [
                pltpu.async_copy(
                    table_hbm.at[idx_v.at[j]],
                    rows_v.at[pl.ds(j * IDX_MINOR, IDX_MINOR)],
                    sem,
                )
                for j in range(rows2)
            ]

            for j in range(rows2):
                def mask_body(i, _, j=j):
                    v = idx_v[j, pl.ds(i * L, L)]
                    mask_v[j, pl.ds(i * L, L)] = jnp.minimum(
                        jnp.abs(v), 1
                    ).astype(jnp.float32)
                    return _

                lax.fori_loop(0, IDX_MINOR // L, mask_body, 0)

            for cp in copies:
                cp.wait()
            pltpu.sync_copy(rows_v, emb_hbm.at[pl.ds(base, CHUNK)])
            pltpu.sync_copy(mask_v, mask_hbm.at[pl.ds(rb, rows2)])

    return emb_kernel


def kernel(x, table):
    bs, sl, wl = x.shape
    B = bs * sl * wl
    x2d = x.reshape(B // IDX_MINOR, IDX_MINOR).astype(jnp.int32)
    emb, mask = _build(B)(x2d, table.astype(jnp.float32))
    return emb.reshape(bs, sl, wl, D), mask.reshape(bs, sl, wl)


# speedup vs baseline: 2.2098x; 1.0011x over previous
import functools

import jax
import jax.numpy as jnp
from jax import lax
from jax.experimental import pallas as pl
from jax.experimental.pallas import tpu as pltpu
from jax.experimental.pallas import tpu_sc as plsc

L = 16
IDX_MINOR = 128
CHUNK = 512
NBUF = 3
D = 64


@functools.cache
def _build(B):
    info = plsc.get_sparse_core_info()
    nw = info.num_cores * info.num_subcores
    b_per_w = B // nw
    n_chunks = b_per_w // CHUNK
    rows2 = CHUNK // IDX_MINOR
    mesh = plsc.VectorSubcoreMesh(core_axis_name="c", subcore_axis_name="s")

    @functools.partial(
        pl.kernel,
        mesh=mesh,
        out_type=[
            jax.ShapeDtypeStruct((B, D), jnp.float32),
            jax.ShapeDtypeStruct((B // IDX_MINOR, IDX_MINOR), jnp.float32),
        ],
        scratch_types=[
            [pltpu.VMEM((rows2, IDX_MINOR), jnp.int32) for _ in range(NBUF)],
            [pltpu.VMEM((CHUNK, D), jnp.float32) for _ in range(NBUF)],
            [pltpu.VMEM((rows2, IDX_MINOR), jnp.float32) for _ in range(NBUF)],
            [pltpu.SemaphoreType.DMA for _ in range(NBUF)],
            [pltpu.SemaphoreType.DMA for _ in range(NBUF)],
        ],
        compiler_params=pltpu.CompilerParams(use_tc_tiling_on_sc=False),
    )
    def emb_kernel(x_hbm, table_hbm, emb_hbm, mask_hbm,
                   idx_v, rows_v, mask_v, sem_in, sem_out):
        wid = lax.axis_index("s") * info.num_cores + lax.axis_index("c")
        w_base = wid * b_per_w

        idx_pend = [None] * NBUF
        out_pend = [None] * NBUF

        for b in range(min(NBUF, n_chunks)):
            rb = pl.multiple_of((w_base + b * CHUNK) // IDX_MINOR, rows2)
            idx_pend[b] = pltpu.async_copy(
                x_hbm.at[pl.ds(rb, rows2)], idx_v[b], sem_in[b])

        for t in range(n_chunks):
            b = t % NBUF
            base = pl.multiple_of(w_base + t * CHUNK, CHUNK)
            rb = pl.multiple_of(base // IDX_MINOR, rows2)

            idx_pend[b].wait()
            if out_pend[b] is not None:
                for h in out_pend[b]:
                    h.wait()

            gathers = [
                pltpu.async_copy(
                    table_hbm.at[idx_v[b].at[j]],
                    rows_v[b].at[pl.ds(j * IDX_MINOR, IDX_MINOR)],
                    sem_in[b],
                )
                for j in range(rows2)
            ]

            for j in range(rows2):
                def mask_body(i, _, j=j, b=b):
                    v = idx_v[b][j, pl.ds(i * L, L)]
                    mask_v[b][j, pl.ds(i * L, L)] = jnp.minimum(
                        jnp.abs(v), 1
                    ).astype(jnp.float32)
                    return _

                lax.fori_loop(0, IDX_MINOR // L, mask_body, 0)

            for g in gathers:
                g.wait()

            out_pend[b] = [
                pltpu.async_copy(rows_v[b], emb_hbm.at[pl.ds(base, CHUNK)],
                                 sem_out[b]),
                pltpu.async_copy(mask_v[b], mask_hbm.at[pl.ds(rb, rows2)],
                                 sem_out[b]),
            ]

            tn = t + NBUF
            if tn < n_chunks:
                rbn = pl.multiple_of((w_base + tn * CHUNK) // IDX_MINOR, rows2)
                idx_pend[b] = pltpu.async_copy(
                    x_hbm.at[pl.ds(rbn, rows2)], idx_v[b], sem_in[b])

        for hs in out_pend:
            if hs is not None:
                for h in hs:
                    h.wait()

    return emb_kernel


def kernel(x, table):
    bs, sl, wl = x.shape
    B = bs * sl * wl
    x2d = x.reshape(B // IDX_MINOR, IDX_MINOR).astype(jnp.int32)
    emb, mask = _build(B)(x2d, table.astype(jnp.float32))
    return emb.reshape(bs, sl, wl, D), mask.reshape(bs, sl, wl)
